# baseline (speedup 1.0000x reference)
import functools

import jax
import jax.numpy as jnp
import numpy as np
from jax import lax
from jax.experimental import pallas as pl
from jax.experimental.pallas import tpu as pltpu
from jax.experimental.pallas import tpu_sc as plsc

DEPTH = 11
N_NODES = 2 ** DEPTH - 1
N_PAD = 2 ** DEPTH
D = 512
NBUF = 7


def _layout_np():
    perm = np.zeros(N_PAD, np.int32)
    counts = np.zeros((N_PAD, 1), np.float32)
    cur = np.array([0], np.int32)
    for lev in range(DEPTH):
        off = 2 ** lev
        perm[off:off + off] = cur
        counts[off:off + off, 0] = 2 ** (DEPTH - lev) - 1
        cur = np.concatenate([2 * cur + 1, 2 * cur + 2])
    return perm, counts


_PERM, _COUNTS = _layout_np()


def _make_sc_gather(n_rows, nc, ns, chunk):
    nw = nc * ns
    per_w = n_rows // nw
    nch = per_w // chunk
    ich = 128
    nich = per_w // ich
    mesh = plsc.VectorSubcoreMesh(core_axis_name="c", subcore_axis_name="s")

    @functools.partial(
        pl.kernel,
        mesh=mesh,
        out_type=jax.ShapeDtypeStruct((n_rows, D), jnp.float32),
        scratch_types=(
            [pltpu.VMEM((nich, ich), jnp.int32),
             pltpu.VMEM((per_w,), jnp.int32),
             pltpu.SemaphoreType.DMA]
            + [pltpu.VMEM((chunk, D), jnp.float32) for _ in range(NBUF)]
            + [pltpu.SemaphoreType.DMA for _ in range(2 * NBUF)]
        ),
    )
    def gather_k(pf_hbm, x_hbm, emb_hbm, out_hbm, pf_v, idx_v, isem,
                 *bufs_sems):
        bufs = bufs_sems[:NBUF]
        gsems = bufs_sems[NBUF:2 * NBUF]
        wsems = bufs_sems[2 * NBUF:]
        wid = lax.axis_index("s") * nc + lax.axis_index("c")
        base = wid * per_w
        pltpu.sync_copy(pf_hbm.at[wid], pf_v)
        pend_i = [pltpu.async_copy(x_hbm.at[pf_v.at[q]],
                                   idx_v.at[pl.ds(q * ich, ich)], isem)
                  for q in range(nich)]
        idx_ready = [False] * nich

        def need_idx(i):
            q = (i * chunk) // ich
            if not idx_ready[q]:
                pend_i[q].wait()
                idx_ready[q] = True

        nbuf = NBUF
        pend_g = [None] * nbuf
        pend_w = [None] * nbuf
        for b in range(min(nbuf, nch)):
            need_idx(b)
            pend_g[b] = pltpu.async_copy(
                emb_hbm.at[idx_v.at[pl.ds(b * chunk, chunk)]],
                bufs[b], gsems[b])
        for i in range(nch):
            b = i % nbuf
            pend_g[b].wait()
            pend_w[b] = pltpu.async_copy(
                bufs[b], out_hbm.at[pl.ds(base + i * chunk, chunk)], wsems[b])
            nxt = i + nbuf
            if nxt < nch:
                pend_w[b].wait()
                need_idx(nxt)
                pend_g[b] = pltpu.async_copy(
                    emb_hbm.at[idx_v.at[pl.ds(nxt * chunk, chunk)]],
                    bufs[b], gsems[b])
        for q in range(nich):
            if not idx_ready[q]:
                pend_i[q].wait()
        for i in range(max(0, nch - nbuf), nch):
            pend_w[i % nbuf].wait()

    return gather_k


def _tc_body(cnt_ref, g_ref, w_ref, b_ref, o_ref):
    a = g_ref.at[0]
    for lev in range(DEPTH - 2, -1, -1):
        off = 2 ** lev
        a[pl.ds(off, off), :] += (a[pl.ds(2 * off, off), :]
                                  + a[pl.ds(3 * off, off), :])
    enc = jnp.dot(a[...].astype(jnp.bfloat16), w_ref[...],
                  preferred_element_type=jnp.float32)
    enc = enc + cnt_ref[...] * b_ref[...]
    node = lax.broadcasted_iota(jnp.int32, (N_PAD, 1), 0)
    enc = jnp.where(node > 0, enc, -jnp.inf)
    o_ref[...] = jnp.max(enc, axis=0, keepdims=True)[None]


def _tc_call(counts, g, w, b):
    bs = g.shape[0]
    return pl.pallas_call(
        _tc_body,
        grid=(bs,),
        in_specs=[
            pl.BlockSpec((N_PAD, 1), lambda i: (0, 0)),
            pl.BlockSpec((1, N_PAD, D), lambda i: (i, 0, 0)),
            pl.BlockSpec((D, D), lambda i: (0, 0)),
            pl.BlockSpec((1, D), lambda i: (0, 0)),
        ],
        out_specs=pl.BlockSpec((1, 1, D), lambda i: (i, 0, 0)),
        out_shape=jax.ShapeDtypeStruct((bs, 1, D), jnp.float32),
    )(counts, g, w, b)


def _perm_flat_np(batch, n, nw, nich, ich):
    r = np.arange(batch * N_PAD, dtype=np.int64)
    b = r // N_PAD
    pf = (b * n + _PERM[r % N_PAD].astype(np.int64)).astype(np.int32)
    return pf.reshape(nw, nich, ich)


def kernel(x, bs, emb, W_c, b_c):
    x = x.astype(jnp.int32)
    batch, n = x.shape
    n_rows = batch * N_PAD
    info = plsc.get_sparse_core_info()
    nc, ns = info.num_cores, info.num_subcores
    chunk = 32
    nw = nc * ns
    pf = jnp.asarray(_perm_flat_np(batch, n, nw, (n_rows // nw) // 128, 128))
    gather = _make_sc_gather(n_rows, nc, ns, chunk)
    g = gather(pf, x.reshape(-1), emb)
    counts = jnp.asarray(_COUNTS)
    out = _tc_call(counts, g.reshape(batch, N_PAD, D),
                   W_c.astype(jnp.bfloat16), b_c.reshape(1, D))
    return out.reshape(batch, D)


# speedup vs baseline: 1.0015x; 1.0015x over previous
import functools

import jax
import jax.numpy as jnp
import numpy as np
from jax import lax
from jax.experimental import pallas as pl
from jax.experimental.pallas import tpu as pltpu
from jax.experimental.pallas import tpu_sc as plsc

DEPTH = 11
N_NODES = 2 ** DEPTH - 1
N_PAD = 2 ** DEPTH
D = 512
NBUF = 6


def _layout_np():
    perm = np.zeros(N_PAD, np.int32)
    counts = np.zeros((N_PAD, 1), np.float32)
    cur = np.array([0], np.int32)
    for lev in range(DEPTH):
        off = 2 ** lev
        perm[off:off + off] = cur
        counts[off:off + off, 0] = 2 ** (DEPTH - lev) - 1
        cur = np.concatenate([2 * cur + 1, 2 * cur + 2])
    return perm, counts


_PERM, _COUNTS = _layout_np()


def _make_sc_gather(n_rows, nc, ns, chunk):
    nw = nc * ns
    per_w = n_rows // nw
    nch = per_w // chunk
    ich = 128
    nich = per_w // ich
    mesh = plsc.VectorSubcoreMesh(core_axis_name="c", subcore_axis_name="s")

    @functools.partial(
        pl.kernel,
        mesh=mesh,
        out_type=jax.ShapeDtypeStruct((n_rows, D), jnp.float32),
        scratch_types=(
            [pltpu.VMEM((nich, ich), jnp.int32),
             pltpu.VMEM((per_w,), jnp.int32),
             pltpu.SemaphoreType.DMA]
            + [pltpu.VMEM((chunk, D), jnp.float32) for _ in range(NBUF)]
            + [pltpu.SemaphoreType.DMA for _ in range(2 * NBUF)]
        ),
    )
    def gather_k(pf_hbm, x_hbm, emb_hbm, out_hbm, pf_v, idx_v, isem,
                 *bufs_sems):
        bufs = bufs_sems[:NBUF]
        gsems = bufs_sems[NBUF:2 * NBUF]
        wsems = bufs_sems[2 * NBUF:]
        wid = lax.axis_index("s") * nc + lax.axis_index("c")
        base = wid * per_w
        pltpu.sync_copy(pf_hbm.at[wid], pf_v)
        pend_i = [pltpu.async_copy(x_hbm.at[pf_v.at[q]],
                                   idx_v.at[pl.ds(q * ich, ich)], isem)
                  for q in range(nich)]
        idx_ready = [False] * nich

        def need_idx(i):
            q = (i * chunk) // ich
            if not idx_ready[q]:
                pend_i[q].wait()
                idx_ready[q] = True

        nbuf = NBUF
        pend_g = [None] * nbuf
        pend_w = [None] * nbuf
        for b in range(min(nbuf, nch)):
            need_idx(b)
            pend_g[b] = pltpu.async_copy(
                emb_hbm.at[idx_v.at[pl.ds(b * chunk, chunk)]],
                bufs[b], gsems[b])
        for i in range(nch):
            b = i % nbuf
            pend_g[b].wait()
            pend_w[b] = pltpu.async_copy(
                bufs[b], out_hbm.at[pl.ds(base + i * chunk, chunk)], wsems[b])
            nxt = i + nbuf
            if nxt < nch:
                pend_w[b].wait()
                need_idx(nxt)
                pend_g[b] = pltpu.async_copy(
                    emb_hbm.at[idx_v.at[pl.ds(nxt * chunk, chunk)]],
                    bufs[b], gsems[b])
        for q in range(nich):
            if not idx_ready[q]:
                pend_i[q].wait()
        for i in range(max(0, nch - nbuf), nch):
            pend_w[i % nbuf].wait()

    return gather_k


def _tc_body(cnt_ref, g_ref, w_ref, b_ref, o_ref):
    a = g_ref.at[0]
    for lev in range(DEPTH - 2, -1, -1):
        off = 2 ** lev
        a[pl.ds(off, off), :] += (a[pl.ds(2 * off, off), :]
                                  + a[pl.ds(3 * off, off), :])
    enc = jnp.dot(a[...].astype(jnp.bfloat16), w_ref[...],
                  preferred_element_type=jnp.float32)
    enc = enc + cnt_ref[...] * b_ref[...]
    node = lax.broadcasted_iota(jnp.int32, (N_PAD, 1), 0)
    enc = jnp.where(node > 0, enc, -jnp.inf)
    o_ref[...] = jnp.max(enc, axis=0, keepdims=True)[None]


def _tc_call(counts, g, w, b):
    bs = g.shape[0]
    return pl.pallas_call(
        _tc_body,
        grid=(bs,),
        in_specs=[
            pl.BlockSpec((N_PAD, 1), lambda i: (0, 0)),
            pl.BlockSpec((1, N_PAD, D), lambda i: (i, 0, 0)),
            pl.BlockSpec((D, D), lambda i: (0, 0)),
            pl.BlockSpec((1, D), lambda i: (0, 0)),
        ],
        out_specs=pl.BlockSpec((1, 1, D), lambda i: (i, 0, 0)),
        out_shape=jax.ShapeDtypeStruct((bs, 1, D), jnp.float32),
    )(counts, g, w, b)


def _perm_flat_np(batch, n, nw, nich, ich):
    r = np.arange(batch * N_PAD, dtype=np.int64)
    b = r // N_PAD
    pf = (b * n + _PERM[r % N_PAD].astype(np.int64)).astype(np.int32)
    return pf.reshape(nw, nich, ich)


def kernel(x, bs, emb, W_c, b_c):
    x = x.astype(jnp.int32)
    batch, n = x.shape
    n_rows = batch * N_PAD
    info = plsc.get_sparse_core_info()
    nc, ns = info.num_cores, info.num_subcores
    chunk = 32
    nw = nc * ns
    pf = jnp.asarray(_perm_flat_np(batch, n, nw, (n_rows // nw) // 128, 128))
    gather = _make_sc_gather(n_rows, nc, ns, chunk)
    g = gather(pf, x.reshape(-1), emb)
    counts = jnp.asarray(_COUNTS)
    out = _tc_call(counts, g.reshape(batch, N_PAD, D),
                   W_c.astype(jnp.bfloat16), b_c.reshape(1, D))
    return out.reshape(batch, D)
